# initial kernel scaffold (unmeasured)
import jax
import jax.numpy as jnp
from jax import lax
from jax.experimental import pallas as pl
from jax.experimental.pallas import tpu as pltpu

N_EXP_LOCAL = 4
CHUNK = 512


def kernel(x, assign, W1, W2):
    t, d = x.shape
    e_loc, _, f = W1.shape
    assign2 = assign.reshape(t, 1)

    def body(x_ref, a_ref, w1_ref, w2_ref, out_ref,
             px_ref, pa_ref, sbuf_ref, rbuf_ref, ssems, rsems):
        my_x = lax.axis_index("x")
        my_y = lax.axis_index("y")
        my_z = lax.axis_index("z")
        peer = (my_x, 1 - my_y, my_z)

        bsem = pltpu.get_barrier_semaphore()
        pl.semaphore_signal(bsem, inc=1, device_id=peer,
                            device_id_type=pl.DeviceIdType.MESH)
        pl.semaphore_wait(bsem, 1)

        rdma_x = pltpu.make_async_remote_copy(
            src_ref=x_ref, dst_ref=px_ref,
            send_sem=ssems.at[0], recv_sem=rsems.at[0],
            device_id=peer, device_id_type=pl.DeviceIdType.MESH)
        rdma_x.start()
        rdma_a = pltpu.make_async_remote_copy(
            src_ref=a_ref, dst_ref=pa_ref,
            send_sem=ssems.at[1], recv_sem=rsems.at[1],
            device_id=peer, device_id_type=pl.DeviceIdType.MESH)
        rdma_a.start()

        e0 = my_y * N_EXP_LOCAL

        def ffn(src_x_ref, src_a_ref, dst_ref):
            for c0 in range(0, t, CHUNK):
                xc = src_x_ref[pl.ds(c0, CHUNK), :]
                ac = src_a_ref[pl.ds(c0, CHUNK), :]
                acc = jnp.zeros((CHUNK, d), jnp.float32)
                for e in range(N_EXP_LOCAL):
                    h = jnp.maximum(
                        jnp.dot(xc, w1_ref[e],
                                preferred_element_type=jnp.float32), 0.0)
                    y = jnp.dot(h, w2_ref[e],
                                preferred_element_type=jnp.float32)
                    acc = acc + jnp.where(ac == e0 + e, y, 0.0)
                dst_ref[pl.ds(c0, CHUNK), :] = acc

        ffn(x_ref, a_ref, out_ref)

        rdma_x.wait_recv()
        rdma_a.wait_recv()
        ffn(px_ref, pa_ref, sbuf_ref)

        rdma_p = pltpu.make_async_remote_copy(
            src_ref=sbuf_ref, dst_ref=rbuf_ref,
            send_sem=ssems.at[2], recv_sem=rsems.at[2],
            device_id=peer, device_id_type=pl.DeviceIdType.MESH)
        rdma_p.start()
        rdma_p.wait_recv()
        out_ref[...] = out_ref[...] + rbuf_ref[...]

        rdma_x.wait_send()
        rdma_a.wait_send()
        rdma_p.wait_send()

    return pl.pallas_call(
        body,
        out_shape=jax.ShapeDtypeStruct((t, d), jnp.float32),
        in_specs=[pl.BlockSpec(memory_space=pltpu.VMEM)] * 4,
        out_specs=pl.BlockSpec(memory_space=pltpu.VMEM),
        scratch_shapes=[
            pltpu.VMEM((t, d), jnp.float32),
            pltpu.VMEM((t, 1), jnp.int32),
            pltpu.VMEM((t, d), jnp.float32),
            pltpu.VMEM((t, d), jnp.float32),
            pltpu.SemaphoreType.DMA((3,)),
            pltpu.SemaphoreType.DMA((3,)),
        ],
        compiler_params=pltpu.CompilerParams(collective_id=0),
    )(x, assign2, W1, W2)


# baseline (device time: 288548 ns/iter reference)
import jax
import jax.numpy as jnp
from jax import lax
from jax.experimental import pallas as pl
from jax.experimental.pallas import tpu as pltpu

N_EXP_LOCAL = 4
CHUNK = 512


def kernel(x, assign, W1, W2):
    t, d = x.shape
    e_loc, _, f = W1.shape
    xb = x.astype(jnp.bfloat16)
    W1b = W1.astype(jnp.bfloat16)
    W2b = W2.astype(jnp.bfloat16)
    assign2 = assign.reshape(t, 1)

    def body(x_ref, a_ref, w1_ref, w2_ref, out_ref,
             px_ref, pa_ref, sbuf_ref, rbuf_ref, w1b_ref, w2b_ref,
             ssems, rsems, wsems):
        my_x = lax.axis_index("x")
        my_y = lax.axis_index("y")
        my_z = lax.axis_index("z")
        peer = (my_x, 1 - my_y, my_z)

        bsem = pltpu.get_barrier_semaphore()
        pl.semaphore_signal(bsem, inc=1, device_id=peer,
                            device_id_type=pl.DeviceIdType.MESH)
        pl.semaphore_wait(bsem, 1)

        rdma_x = pltpu.make_async_remote_copy(
            src_ref=x_ref, dst_ref=px_ref,
            send_sem=ssems.at[0], recv_sem=rsems.at[0],
            device_id=peer, device_id_type=pl.DeviceIdType.MESH)
        rdma_x.start()
        rdma_a = pltpu.make_async_remote_copy(
            src_ref=a_ref, dst_ref=pa_ref,
            send_sem=ssems.at[1], recv_sem=rsems.at[1],
            device_id=peer, device_id_type=pl.DeviceIdType.MESH)
        rdma_a.start()

        def ffn_expert(e, src_x_ref, src_a_ref, dst_ref, dst_f32):
            e_glob = my_y * N_EXP_LOCAL + e
            for c0 in range(0, t, CHUNK):
                xc = src_x_ref[pl.ds(c0, CHUNK), :]
                ac = src_a_ref[pl.ds(c0, CHUNK), :]
                h = jnp.maximum(
                    jnp.dot(xc, w1b_ref[...],
                            preferred_element_type=jnp.float32), 0.0)
                y = jnp.dot(h.astype(jnp.bfloat16), w2b_ref[...],
                            preferred_element_type=jnp.float32)
                contrib = jnp.where(ac == e_glob, y, 0.0)
                if not dst_f32:
                    contrib = contrib.astype(jnp.bfloat16)
                if e == 0:
                    dst_ref[pl.ds(c0, CHUNK), :] = contrib
                else:
                    dst_ref[pl.ds(c0, CHUNK), :] = (
                        dst_ref[pl.ds(c0, CHUNK), :] + contrib)

        for e in range(N_EXP_LOCAL):
            dma_w1 = pltpu.make_async_copy(w1_ref.at[e], w1b_ref, wsems.at[0])
            dma_w2 = pltpu.make_async_copy(w2_ref.at[e], w2b_ref, wsems.at[1])
            dma_w1.start()
            dma_w2.start()
            dma_w1.wait()
            dma_w2.wait()
            ffn_expert(e, x_ref, a_ref, out_ref, dst_f32=True)
            if e == 0:
                rdma_x.wait_recv()
                rdma_a.wait_recv()
            ffn_expert(e, px_ref, pa_ref, sbuf_ref, dst_f32=False)

        rdma_p = pltpu.make_async_remote_copy(
            src_ref=sbuf_ref, dst_ref=rbuf_ref,
            send_sem=ssems.at[2], recv_sem=rsems.at[2],
            device_id=peer, device_id_type=pl.DeviceIdType.MESH)
        rdma_p.start()
        rdma_p.wait_recv()
        out_ref[...] = out_ref[...] + rbuf_ref[...].astype(jnp.float32)

        rdma_x.wait_send()
        rdma_a.wait_send()
        rdma_p.wait_send()

    return pl.pallas_call(
        body,
        out_shape=jax.ShapeDtypeStruct((t, d), jnp.float32),
        in_specs=[
            pl.BlockSpec(memory_space=pltpu.VMEM),
            pl.BlockSpec(memory_space=pltpu.VMEM),
            pl.BlockSpec(memory_space=pl.ANY),
            pl.BlockSpec(memory_space=pl.ANY),
        ],
        out_specs=pl.BlockSpec(memory_space=pltpu.VMEM),
        scratch_shapes=[
            pltpu.VMEM((t, d), jnp.bfloat16),
            pltpu.VMEM((t, 1), jnp.int32),
            pltpu.VMEM((t, d), jnp.bfloat16),
            pltpu.VMEM((t, d), jnp.bfloat16),
            pltpu.VMEM((d, f), jnp.bfloat16),
            pltpu.VMEM((f, d), jnp.bfloat16),
            pltpu.SemaphoreType.DMA((3,)),
            pltpu.SemaphoreType.DMA((3,)),
            pltpu.SemaphoreType.DMA((2,)),
        ],
        compiler_params=pltpu.CompilerParams(collective_id=0),
    )(xb, assign2, W1b, W2b)


# device time: 243528 ns/iter; 1.1849x vs baseline; 1.1849x over previous
import jax
import jax.numpy as jnp
from jax import lax
from jax.experimental import pallas as pl
from jax.experimental.pallas import tpu as pltpu

N_EXP_LOCAL = 4
CHUNK = 512


def kernel(x, assign, W1, W2):
    t, d = x.shape
    e_loc, _, f = W1.shape
    nch = t // CHUNK
    xb = x.astype(jnp.bfloat16)
    W1b = W1.astype(jnp.bfloat16)
    W2b = W2.astype(jnp.bfloat16)
    assign2 = assign.reshape(t, 1)

    def body(x_ref, a_ref, w1_ref, w2_ref, out_ref,
             px_ref, pa_ref, sbuf_ref, rbuf_ref, w1db_ref, w2db_ref,
             xsends, xrecvs, asends, arecvs, psends, precvs, wsems):
        my_x = lax.axis_index("x")
        my_y = lax.axis_index("y")
        my_z = lax.axis_index("z")
        peer = (my_x, 1 - my_y, my_z)

        bsem = pltpu.get_barrier_semaphore()
        pl.semaphore_signal(bsem, inc=1, device_id=peer,
                            device_id_type=pl.DeviceIdType.MESH)
        pl.semaphore_wait(bsem, 1)

        rdma_a = pltpu.make_async_remote_copy(
            src_ref=a_ref, dst_ref=pa_ref,
            send_sem=asends.at[0], recv_sem=arecvs.at[0],
            device_id=peer, device_id_type=pl.DeviceIdType.MESH)
        rdma_a.start()
        x_rdmas = []
        for c in range(nch):
            r = pltpu.make_async_remote_copy(
                src_ref=x_ref.at[pl.ds(c * CHUNK, CHUNK), :],
                dst_ref=px_ref.at[pl.ds(c * CHUNK, CHUNK), :],
                send_sem=xsends.at[c], recv_sem=xrecvs.at[c],
                device_id=peer, device_id_type=pl.DeviceIdType.MESH)
            r.start()
            x_rdmas.append(r)

        def chunk_compute(e, slot, src_x_ref, src_a_ref, dst_ref, c,
                          to_bf16, first):
            e_glob = my_y * N_EXP_LOCAL + e
            c0 = c * CHUNK
            xc = src_x_ref[pl.ds(c0, CHUNK), :]
            ac = src_a_ref[pl.ds(c0, CHUNK), :]
            h = jnp.maximum(
                jnp.dot(xc, w1db_ref[slot],
                        preferred_element_type=jnp.float32), 0.0)
            y = jnp.dot(h.astype(jnp.bfloat16), w2db_ref[slot],
                        preferred_element_type=jnp.float32)
            contrib = jnp.where(ac == e_glob, y, 0.0)
            if to_bf16:
                contrib = contrib.astype(jnp.bfloat16)
            if first:
                dst_ref[pl.ds(c0, CHUNK), :] = contrib
            else:
                dst_ref[pl.ds(c0, CHUNK), :] = (
                    dst_ref[pl.ds(c0, CHUNK), :] + contrib)

        def all_chunks(e, slot, src_x_ref, src_a_ref, dst_ref, to_bf16,
                       first):
            def fbody(c, carry):
                chunk_compute(e, slot, src_x_ref, src_a_ref, dst_ref, c,
                              to_bf16, first)
                return carry
            lax.fori_loop(0, nch, fbody, 0)

        p_rdmas = []
        for e in range(N_EXP_LOCAL):
            slot = 0
            d1 = pltpu.make_async_copy(w1_ref.at[e], w1db_ref.at[slot],
                                       wsems.at[0])
            d2 = pltpu.make_async_copy(w2_ref.at[e], w2db_ref.at[slot],
                                       wsems.at[1])
            d1.start()
            d2.start()
            d1.wait()
            d2.wait()
            if e == 0:
                all_chunks(e, slot, x_ref, a_ref, out_ref,
                           to_bf16=False, first=True)
                rdma_a.wait_recv()
                for c in range(nch):
                    x_rdmas[c].wait_recv()
                    chunk_compute(e, slot, px_ref, pa_ref, sbuf_ref, c,
                                  to_bf16=True, first=True)
            elif e < N_EXP_LOCAL - 1:
                all_chunks(e, slot, x_ref, a_ref, out_ref,
                           to_bf16=False, first=False)
                all_chunks(e, slot, px_ref, pa_ref, sbuf_ref,
                           to_bf16=True, first=False)
            else:
                for c in range(nch):
                    chunk_compute(e, slot, px_ref, pa_ref, sbuf_ref, c,
                                  to_bf16=True, first=False)
                    r = pltpu.make_async_remote_copy(
                        src_ref=sbuf_ref.at[pl.ds(c * CHUNK, CHUNK), :],
                        dst_ref=rbuf_ref.at[pl.ds(c * CHUNK, CHUNK), :],
                        send_sem=psends.at[c], recv_sem=precvs.at[c],
                        device_id=peer,
                        device_id_type=pl.DeviceIdType.MESH)
                    r.start()
                    p_rdmas.append(r)
                all_chunks(e, slot, x_ref, a_ref, out_ref,
                           to_bf16=False, first=False)

        for r in p_rdmas:
            r.wait_recv()
        out_ref[...] = out_ref[...] + rbuf_ref[...].astype(jnp.float32)

        rdma_a.wait_send()
        for r in x_rdmas:
            r.wait_send()
        for r in p_rdmas:
            r.wait_send()

    return pl.pallas_call(
        body,
        out_shape=jax.ShapeDtypeStruct((t, d), jnp.float32),
        in_specs=[
            pl.BlockSpec(memory_space=pltpu.VMEM),
            pl.BlockSpec(memory_space=pltpu.VMEM),
            pl.BlockSpec(memory_space=pl.ANY),
            pl.BlockSpec(memory_space=pl.ANY),
        ],
        out_specs=pl.BlockSpec(memory_space=pltpu.VMEM),
        scratch_shapes=[
            pltpu.VMEM((t, d), jnp.bfloat16),
            pltpu.VMEM((t, 1), jnp.int32),
            pltpu.VMEM((t, d), jnp.bfloat16),
            pltpu.VMEM((t, d), jnp.bfloat16),
            pltpu.VMEM((1, d, f), jnp.bfloat16),
            pltpu.VMEM((1, f, d), jnp.bfloat16),
            pltpu.SemaphoreType.DMA((4,)),
            pltpu.SemaphoreType.DMA((4,)),
            pltpu.SemaphoreType.DMA((1,)),
            pltpu.SemaphoreType.DMA((1,)),
            pltpu.SemaphoreType.DMA((4,)),
            pltpu.SemaphoreType.DMA((4,)),
            pltpu.SemaphoreType.DMA((2,)),
        ],
        compiler_params=pltpu.CompilerParams(collective_id=0),
    )(xb, assign2, W1b, W2b)
